# baseline (speedup 1.0000x reference)
import functools

import jax
import jax.numpy as jnp
from jax import lax
from jax.experimental import pallas as pl
from jax.experimental.pallas import tpu as pltpu
from jax.experimental.pallas import tpu_sc as plsc

N = 10000
E = 320000
NC = 2
NS = 16
NW = NC * NS
EPT = E // NW
C = 80
NCH = EPT // C
NBUF = 3
NRING = (NCH - 2) // NBUF
DC = 80
DNCH = EPT // DC
ROWS_PER = N // NS
RSUB = 25
NPAD = 16 * 640
DEG_ROWS = NPAD // NS

_MESH = plsc.VectorSubcoreMesh(core_axis_name="c", subcore_axis_name="s")
_SC_PARAMS = pltpu.CompilerParams(use_tc_tiling_on_sc=False)


def _deg_body(dst3, p0, p1, dstv, ones_v, dbounce, acc):
    c = lax.axis_index("c")
    s = lax.axis_index("s")
    w = c * NS + s
    pltpu.sync_copy(dst3.at[w], dstv)
    for i in range(DC // 16):
        ones_v[pl.ds(i * 16, 16)] = jnp.ones((16,), jnp.float32)
    for i in range(DEG_ROWS // 16):
        dbounce[pl.ds(i * 16, 16)] = jnp.zeros((16,), jnp.float32)
    r0 = s * DEG_ROWS
    pltpu.sync_copy(dbounce, acc.at[pl.ds(r0, DEG_ROWS)])
    plsc.subcore_barrier()

    def chunk(j, carry):
        pltpu.sync_copy(ones_v, acc.at[dstv.at[j]], add=True)
        return carry

    lax.fori_loop(0, DNCH, chunk, 0)
    plsc.subcore_barrier()
    pltpu.sync_copy(acc.at[pl.ds(r0, DEG_ROWS)], dbounce)

    @pl.when(c == 0)
    def _():
        pltpu.sync_copy(dbounce, p0.at[pl.ds(r0, DEG_ROWS)])

    @pl.when(c == 1)
    def _():
        pltpu.sync_copy(dbounce, p1.at[pl.ds(r0, DEG_ROWS)])


_deg_call = pl.kernel(
    _deg_body,
    out_type=(jax.ShapeDtypeStruct((NPAD,), jnp.float32),
              jax.ShapeDtypeStruct((NPAD,), jnp.float32)),
    mesh=_MESH,
    scratch_types=[
        pltpu.VMEM((DNCH, DC), jnp.int32),
        pltpu.VMEM((DC,), jnp.float32),
        pltpu.VMEM((DEG_ROWS,), jnp.float32),
        pltpu.VMEM_SHARED((NPAD,), jnp.float32),
    ],
    compiler_params=_SC_PARAMS,
)


_BCH = [(0, NBUF * C), (NBUF * C, NBUF * C), (2 * NBUF * C, ROWS_PER - 2 * NBUF * C)]


def _agg_body(d, hp, src3, dst3, p0, p1, srcv, dstv, rows, acc, sem, sem2):
    c = lax.axis_index("c")
    s = lax.axis_index("s")
    w = c * NS + s
    pltpu.sync_copy(src3.at[w], srcv)
    pltpu.sync_copy(dst3.at[w], dstv)
    r0 = s * ROWS_PER

    @pl.when(c == 0)
    def _():
        for off, ln in _BCH:
            pltpu.sync_copy(hp.at[pl.ds(r0 + off, ln)], rows.at[pl.ds(0, ln)])
            pltpu.sync_copy(rows.at[pl.ds(0, ln)], acc.at[pl.ds(r0 + off, ln)])

    @pl.when(c == 1)
    def _():
        def zrow(r, carry):
            for k in range(d // 16):
                rows[r, pl.ds(k * 16, 16)] = jnp.zeros((16,), jnp.float32)
            return carry

        lax.fori_loop(0, NBUF * C, zrow, 0)
        for off, ln in _BCH:
            pltpu.sync_copy(rows.at[pl.ds(0, ln)], acc.at[pl.ds(r0 + off, ln)])

    plsc.subcore_barrier()

    def slot(j, b):
        buf = rows.at[pl.ds(b * C, C)]
        pltpu.make_async_copy(hp.at[srcv.at[j]], buf, sem.at[b]).wait()
        pltpu.async_copy(buf, acc.at[dstv.at[j]], sem2.at[b], add=True)
        jn = j + NBUF - 1
        bp = (b - 1) % NBUF
        bufp = rows.at[pl.ds(bp * C, C)]

        @pl.when((jn >= NBUF) & (jn < NCH))
        def _():
            pltpu.make_async_copy(bufp, acc.at[dstv.at[jn - NBUF]],
                                  sem2.at[bp]).wait()
            pltpu.async_copy(hp.at[srcv.at[jn]], bufp, sem.at[bp])

    for b in range(NBUF):
        pltpu.async_copy(hp.at[srcv.at[b]], rows.at[pl.ds(b * C, C)], sem.at[b])

    def outer(g, carry):
        base = g * NBUF
        for b in range(NBUF):
            slot(base + b, b)
        return carry

    lax.fori_loop(0, NRING, outer, 0)
    for j in range(NBUF * NRING, NCH):
        slot(j, j % NBUF)
    for j in range(NCH - NBUF, NCH):
        b = j % NBUF
        pltpu.make_async_copy(rows.at[pl.ds(b * C, C)], acc.at[dstv.at[j]],
                              sem2.at[b]).wait()
    plsc.subcore_barrier()

    @pl.when(c == 0)
    def _():
        for off, ln in _BCH:
            pltpu.sync_copy(acc.at[pl.ds(r0 + off, ln)], rows.at[pl.ds(0, ln)])
            pltpu.sync_copy(rows.at[pl.ds(0, ln)], p0.at[pl.ds(r0 + off, ln)])

    @pl.when(c == 1)
    def _():
        for off, ln in _BCH:
            pltpu.sync_copy(acc.at[pl.ds(r0 + off, ln)], rows.at[pl.ds(0, ln)])
            pltpu.sync_copy(rows.at[pl.ds(0, ln)], p1.at[pl.ds(r0 + off, ln)])


def _make_agg(d):
    return pl.kernel(
        functools.partial(_agg_body, d),
        out_type=(jax.ShapeDtypeStruct((N, d), jnp.float32),
                  jax.ShapeDtypeStruct((N, d), jnp.float32)),
        mesh=_MESH,
        scratch_types=[
            pltpu.VMEM((NCH, C), jnp.int32),
            pltpu.VMEM((NCH, C), jnp.int32),
            pltpu.VMEM((NBUF * C, d), jnp.float32),
            pltpu.VMEM_SHARED((N, d), jnp.float32),
            pltpu.SemaphoreType.DMA((NBUF,)),
            pltpu.SemaphoreType.DMA((NBUF,)),
        ],
        compiler_params=_SC_PARAMS,
    )


_agg128 = _make_agg(128)
_agg64 = _make_agg(64)


_RB = 1000


def _m1_body(x_ref, w_ref, h_ref):
    h_ref[...] = jnp.dot(x_ref[...], w_ref[...], preferred_element_type=jnp.float32)


def _t1_body(h_ref, d0_ref, d1_ref, hp_ref, dinv_ref):
    deg = 1.0 + d0_ref[...] + d1_ref[...]
    dinv = lax.rsqrt(deg)
    hp_ref[...] = h_ref[...] * dinv
    dinv_ref[...] = dinv


def _t2_body(p0_ref, p1_ref, dinv_ref, b_ref, w_ref, out_ref):
    dinv = dinv_ref[...]
    h = (p0_ref[...] + p1_ref[...]) * dinv + b_ref[...]
    h = jnp.maximum(h, 0.0)
    out_ref[...] = jnp.dot(h, w_ref[...], preferred_element_type=jnp.float32) * dinv


def _t3_body(q0_ref, q1_ref, dinv_ref, b_ref, out_ref):
    o = (q0_ref[...] + q1_ref[...]) * dinv_ref[...] + b_ref[...]
    m = jnp.max(o, axis=1, keepdims=True)
    lse = m + jnp.log(jnp.sum(jnp.exp(o - m), axis=1, keepdims=True))
    out_ref[...] = o - lse


def _row_spec(d):
    return pl.BlockSpec((_RB, d), lambda i: (i, 0))


def _full_spec(r, c):
    return pl.BlockSpec((r, c), lambda i: (0, 0))


_m1_call = pl.pallas_call(
    _m1_body,
    grid=(N // _RB,),
    in_specs=[_row_spec(128), _full_spec(128, 128)],
    out_specs=_row_spec(128),
    out_shape=jax.ShapeDtypeStruct((N, 128), jnp.float32),
)

_t1_call = pl.pallas_call(
    _t1_body,
    grid=(N // _RB,),
    in_specs=[_row_spec(128), _row_spec(1), _row_spec(1)],
    out_specs=(_row_spec(128), _row_spec(1)),
    out_shape=(jax.ShapeDtypeStruct((N, 128), jnp.float32),
               jax.ShapeDtypeStruct((N, 1), jnp.float32)),
)

_t2_call = pl.pallas_call(
    _t2_body,
    grid=(N // _RB,),
    in_specs=[_row_spec(128), _row_spec(128), _row_spec(1),
              _full_spec(1, 128), _full_spec(128, 64)],
    out_specs=_row_spec(64),
    out_shape=jax.ShapeDtypeStruct((N, 64), jnp.float32),
)

_t3_call = pl.pallas_call(
    _t3_body,
    grid=(N // _RB,),
    in_specs=[_row_spec(64), _row_spec(64), _row_spec(1), _full_spec(1, 64)],
    out_specs=_row_spec(64),
    out_shape=jax.ShapeDtypeStruct((N, 64), jnp.float32),
)


def kernel(x, edge_index, W1, b1, W2, b2):
    src3 = edge_index[0].reshape(NW, NCH, C)
    dst3 = edge_index[1].reshape(NW, NCH, C)

    h1 = _m1_call(x, W1)
    d0, d1 = _deg_call(edge_index[1].reshape(NW, DNCH, DC))
    d0 = d0[:N, None]
    d1 = d1[:N, None]

    hp, dinv = _t1_call(h1, d0, d1)
    p0, p1 = _agg128(hp, src3, dst3)
    h2p = _t2_call(p0, p1, dinv, b1.reshape(1, 128), W2)
    q0, q1 = _agg64(h2p, src3, dst3)
    return _t3_call(q0, q1, dinv, b2.reshape(1, 64))


# speedup vs baseline: 1.0388x; 1.0388x over previous
import functools

import jax
import jax.numpy as jnp
from jax import lax
from jax.experimental import pallas as pl
from jax.experimental.pallas import tpu as pltpu
from jax.experimental.pallas import tpu_sc as plsc

N = 10000
E = 320000
NC = 2
NS = 16
NW = NC * NS
EPT = E // NW
C = 80
NCH = EPT // C
NBUF = 3
NRING = (NCH - 2) // NBUF
DC = 80
DNCH = EPT // DC
ROWS_PER = N // NS
RSUB = 25
NPAD = 16 * 640
DEG_ROWS = NPAD // NS

_MESH = plsc.VectorSubcoreMesh(core_axis_name="c", subcore_axis_name="s")
_SC_PARAMS = pltpu.CompilerParams(use_tc_tiling_on_sc=False)


def _deg_body(dst3, p0, p1, dstv, ones_v, dbounce, acc):
    c = lax.axis_index("c")
    s = lax.axis_index("s")
    w = c * NS + s
    pltpu.sync_copy(dst3.at[w], dstv)
    for i in range(DC // 16):
        ones_v[pl.ds(i * 16, 16)] = jnp.ones((16,), jnp.float32)
    for i in range(DEG_ROWS // 16):
        dbounce[pl.ds(i * 16, 16)] = jnp.zeros((16,), jnp.float32)
    r0 = s * DEG_ROWS
    pltpu.sync_copy(dbounce, acc.at[pl.ds(r0, DEG_ROWS)])
    plsc.subcore_barrier()

    def chunk(j, carry):
        pltpu.sync_copy(ones_v, acc.at[dstv.at[j]], add=True)
        return carry

    lax.fori_loop(0, DNCH, chunk, 0)
    plsc.subcore_barrier()
    pltpu.sync_copy(acc.at[pl.ds(r0, DEG_ROWS)], dbounce)

    @pl.when(c == 0)
    def _():
        pltpu.sync_copy(dbounce, p0.at[pl.ds(r0, DEG_ROWS)])

    @pl.when(c == 1)
    def _():
        pltpu.sync_copy(dbounce, p1.at[pl.ds(r0, DEG_ROWS)])


_deg_call = pl.kernel(
    _deg_body,
    out_type=(jax.ShapeDtypeStruct((NPAD,), jnp.float32),
              jax.ShapeDtypeStruct((NPAD,), jnp.float32)),
    mesh=_MESH,
    scratch_types=[
        pltpu.VMEM((DNCH, DC), jnp.int32),
        pltpu.VMEM((DC,), jnp.float32),
        pltpu.VMEM((DEG_ROWS,), jnp.float32),
        pltpu.VMEM_SHARED((NPAD,), jnp.float32),
    ],
    compiler_params=_SC_PARAMS,
)


_BCH = [(0, NBUF * C), (NBUF * C, NBUF * C), (2 * NBUF * C, ROWS_PER - 2 * NBUF * C)]


def _agg_body(d, hp, src3, dst3, p0, p1, srcv, dstv, rows, acc, sem):
    c = lax.axis_index("c")
    s = lax.axis_index("s")
    w = c * NS + s
    pltpu.sync_copy(src3.at[w], srcv)
    pltpu.sync_copy(dst3.at[w], dstv)
    r0 = s * ROWS_PER

    @pl.when(c == 0)
    def _():
        for off, ln in _BCH:
            pltpu.sync_copy(hp.at[pl.ds(r0 + off, ln)], rows.at[pl.ds(0, ln)])
            pltpu.sync_copy(rows.at[pl.ds(0, ln)], acc.at[pl.ds(r0 + off, ln)])

    @pl.when(c == 1)
    def _():
        def zrow(r, carry):
            for k in range(d // 16):
                rows[r, pl.ds(k * 16, 16)] = jnp.zeros((16,), jnp.float32)
            return carry

        lax.fori_loop(0, NBUF * C, zrow, 0)
        for off, ln in _BCH:
            pltpu.sync_copy(rows.at[pl.ds(0, ln)], acc.at[pl.ds(r0 + off, ln)])

    plsc.subcore_barrier()

    def slot(j, b):
        buf = rows.at[pl.ds(b * C, C)]
        pltpu.make_async_copy(hp.at[srcv.at[j]], buf, sem.at[b]).wait()
        pltpu.sync_copy(buf, acc.at[dstv.at[j]], add=True)

        @pl.when(j + NBUF < NCH)
        def _():
            pltpu.async_copy(hp.at[srcv.at[j + NBUF]], buf, sem.at[b])

    for b in range(NBUF):
        pltpu.async_copy(hp.at[srcv.at[b]], rows.at[pl.ds(b * C, C)], sem.at[b])

    def outer(g, carry):
        base = g * NBUF
        for b in range(NBUF):
            slot(base + b, b)
        return carry

    lax.fori_loop(0, NRING, outer, 0)
    for j in range(NBUF * NRING, NCH):
        slot(j, j % NBUF)
    plsc.subcore_barrier()

    @pl.when(c == 0)
    def _():
        for off, ln in _BCH:
            pltpu.sync_copy(acc.at[pl.ds(r0 + off, ln)], rows.at[pl.ds(0, ln)])
            pltpu.sync_copy(rows.at[pl.ds(0, ln)], p0.at[pl.ds(r0 + off, ln)])

    @pl.when(c == 1)
    def _():
        for off, ln in _BCH:
            pltpu.sync_copy(acc.at[pl.ds(r0 + off, ln)], rows.at[pl.ds(0, ln)])
            pltpu.sync_copy(rows.at[pl.ds(0, ln)], p1.at[pl.ds(r0 + off, ln)])


def _make_agg(d):
    return pl.kernel(
        functools.partial(_agg_body, d),
        out_type=(jax.ShapeDtypeStruct((N, d), jnp.float32),
                  jax.ShapeDtypeStruct((N, d), jnp.float32)),
        mesh=_MESH,
        scratch_types=[
            pltpu.VMEM((NCH, C), jnp.int32),
            pltpu.VMEM((NCH, C), jnp.int32),
            pltpu.VMEM((NBUF * C, d), jnp.float32),
            pltpu.VMEM_SHARED((N, d), jnp.float32),
            pltpu.SemaphoreType.DMA((NBUF,)),
        ],
        compiler_params=_SC_PARAMS,
    )


_agg128 = _make_agg(128)
_agg64 = _make_agg(64)


_RB = 1000


def _m1_body(x_ref, w_ref, h_ref):
    h_ref[...] = jnp.dot(x_ref[...], w_ref[...], preferred_element_type=jnp.float32)


def _t1_body(h_ref, d0_ref, d1_ref, hp_ref, dinv_ref):
    deg = 1.0 + d0_ref[...] + d1_ref[...]
    dinv = lax.rsqrt(deg)
    hp_ref[...] = h_ref[...] * dinv
    dinv_ref[...] = dinv


def _t2_body(p0_ref, p1_ref, dinv_ref, b_ref, w_ref, out_ref):
    dinv = dinv_ref[...]
    h = (p0_ref[...] + p1_ref[...]) * dinv + b_ref[...]
    h = jnp.maximum(h, 0.0)
    out_ref[...] = jnp.dot(h, w_ref[...], preferred_element_type=jnp.float32) * dinv


def _t3_body(q0_ref, q1_ref, dinv_ref, b_ref, out_ref):
    o = (q0_ref[...] + q1_ref[...]) * dinv_ref[...] + b_ref[...]
    m = jnp.max(o, axis=1, keepdims=True)
    lse = m + jnp.log(jnp.sum(jnp.exp(o - m), axis=1, keepdims=True))
    out_ref[...] = o - lse


def _row_spec(d):
    return pl.BlockSpec((_RB, d), lambda i: (i, 0))


def _full_spec(r, c):
    return pl.BlockSpec((r, c), lambda i: (0, 0))


_m1_call = pl.pallas_call(
    _m1_body,
    grid=(N // _RB,),
    in_specs=[_row_spec(128), _full_spec(128, 128)],
    out_specs=_row_spec(128),
    out_shape=jax.ShapeDtypeStruct((N, 128), jnp.float32),
)

_t1_call = pl.pallas_call(
    _t1_body,
    grid=(N // _RB,),
    in_specs=[_row_spec(128), _row_spec(1), _row_spec(1)],
    out_specs=(_row_spec(128), _row_spec(1)),
    out_shape=(jax.ShapeDtypeStruct((N, 128), jnp.float32),
               jax.ShapeDtypeStruct((N, 1), jnp.float32)),
)

_t2_call = pl.pallas_call(
    _t2_body,
    grid=(N // _RB,),
    in_specs=[_row_spec(128), _row_spec(128), _row_spec(1),
              _full_spec(1, 128), _full_spec(128, 64)],
    out_specs=_row_spec(64),
    out_shape=jax.ShapeDtypeStruct((N, 64), jnp.float32),
)

_t3_call = pl.pallas_call(
    _t3_body,
    grid=(N // _RB,),
    in_specs=[_row_spec(64), _row_spec(64), _row_spec(1), _full_spec(1, 64)],
    out_specs=_row_spec(64),
    out_shape=jax.ShapeDtypeStruct((N, 64), jnp.float32),
)


def kernel(x, edge_index, W1, b1, W2, b2):
    src3 = edge_index[0].reshape(NW, NCH, C)
    dst3 = edge_index[1].reshape(NW, NCH, C)

    h1 = _m1_call(x, W1)
    d0, d1 = _deg_call(edge_index[1].reshape(NW, DNCH, DC))
    d0 = d0[:N, None]
    d1 = d1[:N, None]

    hp, dinv = _t1_call(h1, d0, d1)
    p0, p1 = _agg128(hp, src3, dst3)
    h2p = _t2_call(p0, p1, dinv, b1.reshape(1, 128), W2)
    q0, q1 = _agg64(h2p, src3, dst3)
    return _t3_call(q0, q1, dinv, b2.reshape(1, 64))
